# baseline (speedup 1.0000x reference)
import functools

import jax
import jax.numpy as jnp
from jax import lax
from jax.experimental import pallas as pl
from jax.experimental.pallas import tpu as pltpu
from jax.experimental.pallas import tpu_sc as plsc

_B, _H, _W = 8, 512, 512
_R, _C = 2048, 1024
_BR = 256
_NEG_RATIO = 3
_EPS = 1e-06
_MAX_BITS = 0x42C80000


def _stage1(pred_ref, gt_ref, mask_ref, vbits_ref, psum_ref, pcnt_ref,
            ncnt_ref):
    i = pl.program_id(0)
    p = pred_ref[...]
    g = gt_ref[...]
    m = mask_ref[...]
    log_p = jnp.maximum(jnp.log(p), -100.0)
    log_1mp = jnp.maximum(jnp.log(1.0 - p), -100.0)
    loss = -(g * log_p + (1.0 - g) * log_1mp)
    pos = g * m
    neg = (1.0 - g) * m
    v = neg * loss
    vbits_ref[...] = jax.lax.bitcast_convert_type(v, jnp.int32)

    @pl.when(i == 0)
    def _init():
        psum_ref[...] = jnp.zeros((1, 1), jnp.float32)
        pcnt_ref[...] = jnp.zeros((1, 1), jnp.float32)
        ncnt_ref[...] = jnp.zeros((1, 1), jnp.float32)

    psum_ref[...] += jnp.sum(pos * loss, keepdims=True)
    pcnt_ref[...] += jnp.sum(pos, keepdims=True)
    ncnt_ref[...] += jnp.sum(neg, keepdims=True)


def _stage2(k_ref, vbits_ref, tbits_ref, cntgt_ref, sumgt_ref):
    k = k_ref[0]
    bits = vbits_ref[...]

    def body(_, carry):
        lo, hi = carry
        mid = lo + (hi - lo + 1) // 2
        cnt = jnp.sum((bits >= mid).astype(jnp.int32))
        ge = cnt >= k
        return (jnp.where(ge, mid, lo), jnp.where(ge, hi, mid - 1))

    lo, _ = jax.lax.fori_loop(
        0, 31, body, (jnp.int32(0), jnp.int32(_MAX_BITS)))

    vals = jax.lax.bitcast_convert_type(bits, jnp.float32)
    gt_mask = bits > lo
    tbits_ref[...] = jnp.reshape(lo, (1, 1))
    cntgt_ref[...] = jnp.sum(gt_mask.astype(jnp.int32), keepdims=True)
    sumgt_ref[...] = jnp.sum(jnp.where(gt_mask, vals, 0.0), keepdims=True)


_N = _R * _C
_NSUB = 16
_SHARD = _N // _NSUB
_CHUNK = 16384
_NCHUNK = _SHARD // _CHUNK
_H1 = 18432
_S1 = _H1 // 16
_CAP = 16384
_BIG = 0x7FFFFFFF


def _sc_zero_i32(ref, n):
    def body(i, _):
        ref[pl.ds(i * 16, 16)] = jnp.zeros((16,), ref.dtype)
        return 0
    lax.fori_loop(0, n // 16, body, 0)


def _sc_scan(read_vreg, nv, kres, kind=jnp.int32):
    def body(j, carry):
        carry_cnt, nb, cnt_ab, minsuff = carry
        h = read_vreg(nv - 1 - j)
        hr = lax.rev(h, (0,))
        suff = lax.rev(plsc.cumsum(hr), (0,)) + carry_cnt
        m = suff >= kres
        nb = nb + jnp.sum(m.astype(jnp.int32))
        cnt_ab = cnt_ab + jnp.sum(jnp.where(m, jnp.int32(0), h))
        minsuff = jnp.minimum(minsuff, jnp.min(jnp.where(m, suff, _BIG)))
        carry_cnt = carry_cnt + jnp.sum(h)
        return carry_cnt, nb, cnt_ab, minsuff

    _, nb, cnt_ab, minsuff = lax.fori_loop(
        0, nv, body, (jnp.int32(0), jnp.int32(0), jnp.int32(0), jnp.int32(_BIG)))
    return nb - 1, cnt_ab, minsuff


def _sc_body(vbits_hbm, k_hbm, outi_hbm, outf_hbm,
             sbuf, hist1, merged1, cbuf, h2, h3, h3s,
             rbuf2, rbuf3, rbuf3s, m2buf, m3buf, m3sbuf,
             kbuf, tbuf, acc1, obuf_i, obuf_f, xbuf,
             slots1, merged1s, slots2, slots3, slots3s, xchs):
    sid = lax.axis_index("s")
    base = sid * _SHARD
    ones16 = jnp.ones((16,), jnp.int32)

    pltpu.sync_copy(k_hbm, kbuf)
    kvec = kbuf[...]
    k = jnp.max(kvec)

    _sc_zero_i32(hist1, _H1)
    for c in range(_NCHUNK):
        pltpu.sync_copy(vbits_hbm.at[pl.ds(base + c * _CHUNK, _CHUNK)], sbuf)

        def p1(i, _):
            v = sbuf[pl.ds(i * 16, 16)]
            b = lax.shift_right_logical(v, 16)
            plsc.addupdate_scatter(hist1, [b], ones16)
            return 0
        lax.fori_loop(0, _CHUNK // 16, p1, 0)

    pltpu.sync_copy(hist1, slots1.at[pl.ds(sid * _H1, _H1)])
    plsc.subcore_barrier()
    _sc_zero_i32(acc1, _S1)
    for j in range(_NSUB):
        pltpu.sync_copy(slots1.at[pl.ds(j * _H1 + sid * _S1, _S1)], tbuf)

        def radd(i, _):
            acc1[pl.ds(i * 16, 16)] += tbuf[pl.ds(i * 16, 16)]
            return 0
        lax.fori_loop(0, _S1 // 16, radd, 0)
    pltpu.sync_copy(acc1, merged1s.at[pl.ds(sid * _S1, _S1)])
    plsc.subcore_barrier()
    pltpu.sync_copy(merged1s, merged1)

    b1, cnt_ab1, minsuff1 = _sc_scan(
        lambda j: merged1[pl.ds(j * 16, 16)], _H1 // 16, k)
    n1 = minsuff1 - cnt_ab1
    b1v = jnp.zeros((16,), jnp.int32) + b1
    b1f = (b1 <= jnp.int32(0x42C8))

    _sc_zero_i32(h2, 256)

    def p2_chunk(c, carry):
        wp, sacc = carry
        pltpu.sync_copy(vbits_hbm.at[pl.ds(base + c * _CHUNK, _CHUNK)], sbuf)

        def p2(i, carry2):
            wp, sacc = carry2
            v = sbuf[pl.ds(i * 16, 16)]
            b = lax.shift_right_logical(v, 16)
            m_eq = b == b1v
            m_gt = b > b1v
            sacc = sacc + jnp.where(
                m_gt, plsc.bitcast(v, jnp.float32), jnp.float32(0.0))
            b2i = lax.shift_right_logical(v, 8) & 255
            plsc.addupdate_scatter(h2, [b2i], ones16, mask=m_eq)

            @pl.when(wp <= _CAP)
            def _store():
                plsc.store_compressed(cbuf.at[pl.ds(wp, 16)], v, mask=m_eq)
            return wp + jnp.sum(m_eq.astype(jnp.int32)), sacc
        return lax.fori_loop(0, _CHUNK // 16, p2, (wp, sacc))

    wp, sacc = lax.fori_loop(
        0, _NCHUNK, p2_chunk,
        (jnp.int32(0), jnp.zeros((16,), jnp.float32)))

    @pl.when(wp <= _CAP)
    def _sentinel():
        cbuf[pl.ds(wp, 16)] = jnp.zeros((16,), jnp.int32) + _BIG

    pltpu.sync_copy(h2, slots2.at[pl.ds(sid * 256, 256)])
    plsc.subcore_barrier()
    pltpu.sync_copy(slots2, rbuf2)

    def m2m(j, _):
        acc = jnp.zeros((16,), jnp.int32)
        for r in range(_NSUB):
            acc = acc + rbuf2[pl.ds(r * 256 + j * 16, 16)]
        m2buf[pl.ds(j * 16, 16)] = acc
        return 0
    lax.fori_loop(0, 16, m2m, 0)

    k2 = k - cnt_ab1
    b2, cnt_ab2, _ = _sc_scan(lambda j: m2buf[pl.ds(j * 16, 16)], 16, k2)
    b2v = jnp.zeros((16,), jnp.int32) + b2

    _sc_zero_i32(h3, 256)
    _sc_zero_i32(h3s, 256)

    def p3_vregs(src_ref, nvreg, sacc):
        def p3(i, s):
            v = src_ref[pl.ds(i * 16, 16)]
            b = lax.shift_right_logical(v, 16)
            m1 = b == b1v
            b2i = lax.shift_right_logical(v, 8) & 255
            m2 = m1 & (b2i == b2v)
            m2g = m1 & (b2i > b2v)
            s = s + jnp.where(
                m2g, plsc.bitcast(v, jnp.float32), jnp.float32(0.0))
            b3i = v & 255
            plsc.addupdate_scatter(h3, [b3i], ones16, mask=m2)
            plsc.addupdate_scatter(
                h3s, [b3i], plsc.bitcast(v, jnp.float32), mask=m2)
            return s
        return lax.fori_loop(0, nvreg, p3, sacc)

    def p3_fast(sacc):
        return p3_vregs(cbuf, (wp + 15) // 16, sacc)

    def p3_stream(sacc):
        def chunk(c, s):
            pltpu.sync_copy(
                vbits_hbm.at[pl.ds(base + c * _CHUNK, _CHUNK)], sbuf)
            return p3_vregs(sbuf, _CHUNK // 16, s)
        return lax.fori_loop(0, _NCHUNK, chunk, sacc)

    sacc2 = lax.cond(n1 <= _CAP, p3_fast, p3_stream,
                     jnp.zeros((16,), jnp.float32))

    pltpu.sync_copy(h3, slots3.at[pl.ds(sid * 256, 256)])
    pltpu.sync_copy(h3s, slots3s.at[pl.ds(sid * 256, 256)])
    plsc.subcore_barrier()
    pltpu.sync_copy(slots3, rbuf3)
    pltpu.sync_copy(slots3s, rbuf3s)

    def m3m(j, _):
        acc = jnp.zeros((16,), jnp.int32)
        accs = jnp.zeros((16,), jnp.float32)
        for r in range(_NSUB):
            acc = acc + rbuf3[pl.ds(r * 256 + j * 16, 16)]
            accs = accs + rbuf3s[pl.ds(r * 256 + j * 16, 16)]
        m3buf[pl.ds(j * 16, 16)] = acc
        m3sbuf[pl.ds(j * 16, 16)] = accs
        return 0
    lax.fori_loop(0, 16, m3m, 0)

    k3 = k2 - cnt_ab2
    b3, cnt_ab3, _ = _sc_scan(lambda j: m3buf[pl.ds(j * 16, 16)], 16, k3)

    def s3(j, carry):
        cnt_s, sum_s = carry
        bidx = lax.iota(jnp.int32, 16) + j * 16
        mgt = bidx > (jnp.zeros((16,), jnp.int32) + b3)
        sum_s = sum_s + jnp.sum(
            jnp.where(mgt, m3sbuf[pl.ds(j * 16, 16)], jnp.float32(0.0)))
        return cnt_s, sum_s
    _, sum_ab3 = lax.fori_loop(0, 16, s3, (jnp.int32(0), jnp.float32(0.0)))

    obuf_f[...] = sacc + sacc2
    pltpu.sync_copy(obuf_f, xchs.at[pl.ds(sid * 16, 16)])
    plsc.subcore_barrier()
    pltpu.sync_copy(xchs, xbuf)
    acc = jnp.zeros((16,), jnp.float32)
    for r in range(_NSUB):
        acc = acc + xbuf[pl.ds(r * 16, 16)]
    sum_gt = jnp.sum(acc) + sum_ab3

    t_bits = jnp.where(
        b1f,
        lax.shift_left(b1, 16) | lax.shift_left(b2, 8) | b3,
        jnp.int32(0))
    cnt_gt = cnt_ab1 + cnt_ab2 + cnt_ab3

    @pl.when(sid == 0)
    def _out():
        ii = lax.iota(jnp.int32, 16)
        obuf_i[...] = jnp.where(ii == 0, t_bits,
                                jnp.where(ii == 1, cnt_gt, jnp.int32(0)))
        obuf_f[...] = jnp.zeros((16,), jnp.float32) + sum_gt
        pltpu.sync_copy(obuf_i, outi_hbm)
        pltpu.sync_copy(obuf_f, outf_hbm)


def _select_sc(vbits_flat, k):
    mesh = plsc.VectorSubcoreMesh(
        core_axis_name="c", subcore_axis_name="s", num_cores=1)
    kvec = jnp.full((16,), k, jnp.int32)
    f = pl.kernel(
        _sc_body,
        out_type=(jax.ShapeDtypeStruct((16,), jnp.int32),
                  jax.ShapeDtypeStruct((16,), jnp.float32)),
        mesh=mesh,
        compiler_params=pltpu.CompilerParams(needs_layout_passes=False),
        scratch_types=[
            pltpu.VMEM((_CHUNK,), jnp.int32),
            pltpu.VMEM((_H1,), jnp.int32),
            pltpu.VMEM((_H1,), jnp.int32),
            pltpu.VMEM((_CAP + 16,), jnp.int32),
            pltpu.VMEM((256,), jnp.int32),
            pltpu.VMEM((256,), jnp.int32),
            pltpu.VMEM((256,), jnp.float32),
            pltpu.VMEM((_NSUB * 256,), jnp.int32),
            pltpu.VMEM((_NSUB * 256,), jnp.int32),
            pltpu.VMEM((_NSUB * 256,), jnp.float32),
            pltpu.VMEM((256,), jnp.int32),
            pltpu.VMEM((256,), jnp.int32),
            pltpu.VMEM((256,), jnp.float32),
            pltpu.VMEM((16,), jnp.int32),
            pltpu.VMEM((_S1,), jnp.int32),
            pltpu.VMEM((_S1,), jnp.int32),
            pltpu.VMEM((16,), jnp.int32),
            pltpu.VMEM((16,), jnp.float32),
            pltpu.VMEM((_NSUB * 16,), jnp.float32),
            pltpu.VMEM_SHARED((_NSUB * _H1,), jnp.int32),
            pltpu.VMEM_SHARED((_H1,), jnp.int32),
            pltpu.VMEM_SHARED((_NSUB * 256,), jnp.int32),
            pltpu.VMEM_SHARED((_NSUB * 256,), jnp.int32),
            pltpu.VMEM_SHARED((_NSUB * 256,), jnp.float32),
            pltpu.VMEM_SHARED((_NSUB * 16,), jnp.float32),
        ],
    )
    return f(vbits_flat, kvec)


def kernel(pred, gt, mask):
    p2 = pred.reshape(_R, _C)
    g2 = gt.reshape(_R, _C)
    m2 = mask.reshape(_R, _C)

    vbits, psum, pcnt, ncnt = pl.pallas_call(
        _stage1,
        grid=(_R // _BR,),
        in_specs=[pl.BlockSpec((_BR, _C), lambda i: (i, 0))] * 3,
        out_specs=[
            pl.BlockSpec((_BR, _C), lambda i: (i, 0)),
            pl.BlockSpec((1, 1), lambda i: (0, 0)),
            pl.BlockSpec((1, 1), lambda i: (0, 0)),
            pl.BlockSpec((1, 1), lambda i: (0, 0)),
        ],
        out_shape=[
            jax.ShapeDtypeStruct((_R, _C), jnp.int32),
            jax.ShapeDtypeStruct((1, 1), jnp.float32),
            jax.ShapeDtypeStruct((1, 1), jnp.float32),
            jax.ShapeDtypeStruct((1, 1), jnp.float32),
        ],
    )(p2, g2, m2)

    pos_cnt = pcnt[0, 0].astype(jnp.int32)
    neg_cnt = jnp.minimum(
        ncnt[0, 0], (pos_cnt * _NEG_RATIO).astype(jnp.float32)
    ).astype(jnp.int32)

    outi, outf = _select_sc(vbits.reshape(_N), neg_cnt)
    tbits, cntgt, sumgt = outi[0], outi[1], outf[0]

    t = jax.lax.bitcast_convert_type(tbits, jnp.float32)
    neg_top = jnp.where(
        neg_cnt > 0,
        sumgt + (neg_cnt - cntgt).astype(jnp.float32) * t,
        0.0,
    )
    denom = (pos_cnt + neg_cnt).astype(jnp.float32) + _EPS
    return (psum[0, 0] + neg_top) / denom
